# baseline (speedup 1.0000x reference)
import jax
import jax.numpy as jnp
from jax import lax
from jax.experimental import pallas as pl

_N = 128
_BB = 256


def _diag_kernel(x_ref, o_ref):
    i = lax.broadcasted_iota(jnp.int32, (_N, _N), 0)
    j = lax.broadcasted_iota(jnp.int32, (_N, _N), 1)
    mask = (i == j)[None]
    o_ref[...] = jnp.where(mask, x_ref[...][:, :, None], jnp.float32(0))


def kernel(x):
    b = x.shape[0]
    grid = (b // _BB,)
    return pl.pallas_call(
        _diag_kernel,
        grid=grid,
        in_specs=[pl.BlockSpec((_BB, _N), lambda g: (g, 0))],
        out_specs=pl.BlockSpec((_BB, _N, _N), lambda g: (g, 0, 0)),
        out_shape=jax.ShapeDtypeStruct((b, _N, _N), x.dtype),
    )(x)
